# baseline (speedup 1.0000x reference)
import functools

import jax
import jax.numpy as jnp
from jax import lax
from jax.experimental import pallas as pl
from jax.experimental.pallas import tpu as pltpu
from jax.experimental.pallas import tpu_sc as plsc

_NC = 2
_NS = 16
_L = 16
_CHUNK = 128
_FILLS = 2048
_FIX_ROUNDS = 3


def _iota16():
    return lax.broadcasted_iota(jnp.int32, (_L,), 0)


def _make_sc_call(M, D, B, S):
    assert D == 64
    assert B % (_NS * _CHUNK) == 0
    assert S % (_NC * _NS * _CHUNK) == 0
    wpt = B // _NS
    wk = wpt // _CHUNK
    spw = S // (_NC * _NS)
    sk = spw // _CHUNK
    span = ((M + _NS * _FILLS - 1) // (_NS * _FILLS)) * _FILLS
    dummy = _NS * span
    table_n = dummy + _L

    mesh = plsc.VectorSubcoreMesh(core_axis_name="c", subcore_axis_name="s")

    @functools.partial(
        pl.kernel,
        mesh=mesh,
        out_type=jax.ShapeDtypeStruct((S // 2, 2 * D), jnp.float32),
        scratch_types=[
            pltpu.VMEM_SHARED((table_n,), jnp.int32),
            pltpu.VMEM((_FILLS,), jnp.int32),
            pltpu.VMEM((wk, _CHUNK), jnp.int32),
            pltpu.VMEM((wk, _CHUNK), jnp.int32),
            pltpu.VMEM((wk, _CHUNK), jnp.int32),
            pltpu.VMEM((wk, _CHUNK), jnp.int32),
            pltpu.VMEM((_CHUNK,), jnp.int32),
            pltpu.VMEM((_CHUNK,), jnp.int32),
            pltpu.VMEM((_CHUNK,), jnp.int32),
            pltpu.VMEM((_CHUNK,), jnp.int32),
            pltpu.VMEM((_CHUNK,), jnp.int32),
            pltpu.VMEM((_CHUNK,), jnp.int32),
            pltpu.VMEM((_CHUNK,), jnp.int32),
            pltpu.VMEM((_CHUNK,), jnp.float32),
            pltpu.VMEM((_CHUNK,), jnp.float32),
            pltpu.VMEM((_CHUNK,), jnp.float32),
            pltpu.VMEM((_CHUNK, 2 * D), jnp.float32),
            pltpu.VMEM((_CHUNK, 2 * D), jnp.float32),
            pltpu.VMEM((_CHUNK // 2, 2 * D), jnp.float32),
            pltpu.SemaphoreType.DMA,
            pltpu.SemaphoreType.DMA,
            pltpu.SemaphoreType.DMA,
        ],
        compiler_params=pltpu.CompilerParams(
            needs_layout_passes=False, use_tc_tiling_on_sc=False),
    )
    def sc_call(mem2, mimp, feats2, fimp, widx_h, sidx_h, out2,
                table, fillv, widx_v, jval_v, tv, fixidx,
                srow, prow, rarow, jwrow, idxbrow, pjrow, rbrow,
                impa, impb, wrow, mema, featb, outbuf,
                sem_a, sem_b, sem_f):
        c = lax.axis_index("c")
        s = lax.axis_index("s")
        wid = c * _NS + s
        i16 = _iota16()

        for q in range(_FILLS // _L):
            fillv[pl.ds(q * _L, _L)] = jnp.full((_L,), -1, jnp.int32)
        base = s * span
        ms = [pltpu.async_copy(fillv, table.at[pl.ds(base + q * _FILLS, _FILLS)],
                               sem_a)
              for q in range(span // _FILLS)]

        pltpu.sync_copy(widx_h.at[s], widx_v)
        jbase = s * wpt
        for k in range(wk):
            for v in range(_CHUNK // _L):
                jval_v[k, pl.ds(v * _L, _L)] = i16 + (jbase + k * _CHUNK + v * _L)
        for cp in ms:
            cp.wait()

        plsc.subcore_barrier()

        sc0 = [pltpu.async_copy(jval_v.at[k], table.at[widx_v.at[k]], sem_a)
               for k in range(wk)]
        for cp in sc0:
            cp.wait()

        for _r in range(_FIX_ROUNDS):
            plsc.subcore_barrier()
            gt = [pltpu.async_copy(table.at[widx_v.at[k]], tv.at[k], sem_a)
                  for k in range(wk)]
            for cp in gt:
                cp.wait()
            for k in range(wk):
                for v in range(_CHUNK // _L):
                    sl = pl.ds(v * _L, _L)
                    jv = jval_v[k, sl]
                    fixidx[k, sl] = jnp.where(tv[k, sl] < jv, widx_v[k, sl],
                                              dummy)
            st = [pltpu.async_copy(jval_v.at[k], table.at[fixidx.at[k]], sem_a)
                  for k in range(wk)]
            for cp in st:
                cp.wait()

        plsc.subcore_barrier()

        def chunk_body(k, _):
            pltpu.sync_copy(sidx_h.at[wid, k], srow)
            for q in range(_CHUNK // _L):
                sl = pl.ds(q * _L, _L)
                sv = srow[sl]
                prow[sl] = lax.shift_right_logical(sv, 1)
                rarow[sl] = lax.shift_left(jnp.bitwise_and(sv, 1), 6)
            cpa = pltpu.async_copy(mem2.at[prow], mema, sem_a)
            cpi = pltpu.async_copy(mimp.at[srow], impa, sem_f)
            pltpu.sync_copy(table.at[srow], jwrow)
            for q in range(_CHUNK // _L):
                sl = pl.ds(q * _L, _L)
                jw = jwrow[sl]
                jb = jnp.where(jw >= 0, jw, 0)
                idxbrow[sl] = jb
                pjrow[sl] = lax.shift_right_logical(jb, 1)
                rbrow[sl] = lax.shift_left(jnp.bitwise_and(jb, 1), 6)
            cpb = pltpu.async_copy(feats2.at[pjrow], featb, sem_b)
            cpj = pltpu.async_copy(fimp.at[idxbrow], impb, sem_f)
            cpi.wait()
            cpj.wait()
            for q in range(_CHUNK // _L):
                sl = pl.ds(q * _L, _L)
                wrow[sl] = jnp.where(jwrow[sl] >= 0, impb[sl], impa[sl])
            cpa.wait()
            cpb.wait()
            for g in range(_CHUNK // _L):
                sl = pl.ds(g * _L, _L)
                sampvec = i16 + g * _L
                s2v = lax.shift_right_logical(sampvec, 1)
                h64v = lax.shift_left(jnp.bitwise_and(sampvec, 1), 6)
                rav = rarow[sl]
                rbv = rbrow[sl]
                hitv = jwrow[sl] >= 0
                wv = wrow[sl]
                for cc in range(D):
                    a = plsc.load_gather(mema, [sampvec, rav + cc])
                    b = plsc.load_gather(featb, [sampvec, rbv + cc])
                    o = jnp.where(hitv, b, a) * wv
                    plsc.store_scatter(outbuf, [s2v, h64v + cc], o)
            pltpu.sync_copy(outbuf,
                            out2.at[pl.ds(wid * (spw // 2) + k * (_CHUNK // 2),
                                          _CHUNK // 2)])
            return _
        lax.fori_loop(0, sk, chunk_body, None)

    return sc_call


def kernel(memory_features, memory_importance, features, importance,
           write_idx, sample_idx):
    M, D = memory_features.shape
    B = write_idx.shape[0]
    S = sample_idx.shape[0]
    call = _make_sc_call(M, D, B, S)
    mem2 = memory_features.reshape(M // 2, 2 * D)
    feats2 = features.reshape(B // 2, 2 * D)
    widx3 = write_idx.reshape(_NS, B // (_NS * _CHUNK), _CHUNK)
    sidx3 = sample_idx.reshape(_NC * _NS, S // (_NC * _NS * _CHUNK), _CHUNK)
    out2 = call(mem2, memory_importance, feats2, importance, widx3, sidx3)
    return out2.reshape(S, D)


# speedup vs baseline: 1.3796x; 1.3796x over previous
import functools

import jax
import jax.numpy as jnp
from jax import lax
from jax.experimental import pallas as pl
from jax.experimental.pallas import tpu as pltpu
from jax.experimental.pallas import tpu_sc as plsc

_NC = 2
_NS = 16
_L = 16
_CHUNK = 128
_FILLS = 1024
_FIX_ROUNDS = 3

_PARAMS = pltpu.CompilerParams(
    needs_layout_passes=False, use_tc_tiling_on_sc=False)


def _iota16():
    return lax.broadcasted_iota(jnp.int32, (_L,), 0)


def _splat(x):
    return jnp.full((_L,), x, jnp.int32)


def _make_stage1(M, D, B, S):
    wpt = B // _NS
    wk = wpt // _CHUNK
    spw = S // (_NC * _NS)
    sk = spw // _CHUNK
    span = ((M + _NS * _FILLS - 1) // (_NS * _FILLS)) * _FILLS
    dummy = _NS * span
    table_n = dummy + _L

    mesh = plsc.VectorSubcoreMesh(core_axis_name="c", subcore_axis_name="s")

    @functools.partial(
        pl.kernel,
        mesh=mesh,
        out_type=(jax.ShapeDtypeStruct((S, D), jnp.float32),
                  jax.ShapeDtypeStruct((S, D), jnp.float32)),
        scratch_types=[
            pltpu.VMEM_SHARED((table_n,), jnp.int32),
            pltpu.VMEM((_FILLS,), jnp.int32),
            pltpu.VMEM((wk, _CHUNK), jnp.int32),
            pltpu.VMEM((wk, _CHUNK), jnp.int32),
            pltpu.VMEM((_CHUNK,), jnp.int32),
            pltpu.VMEM((_CHUNK,), jnp.int32),
            pltpu.VMEM((_CHUNK,), jnp.int32),
            pltpu.VMEM((_CHUNK,), jnp.int32),
            pltpu.VMEM((_CHUNK,), jnp.int32),
            pltpu.VMEM((_CHUNK,), jnp.float32),
            pltpu.VMEM((_CHUNK,), jnp.float32),
            pltpu.VMEM((_CHUNK,), jnp.float32),
            pltpu.VMEM((_CHUNK,), jnp.float32),
            pltpu.VMEM((_CHUNK, D), jnp.float32),
            pltpu.VMEM((_CHUNK, D), jnp.float32),
            pltpu.VMEM((_CHUNK, D), jnp.float32),
        ],
        compiler_params=_PARAMS,
    )
    def stage1(mimp, feats, fimp, widx_h, sidx_h, hit_h, wexp_h,
               table, fillv, widx_v, jval_v, tvrow, fixrow,
               srow, jwrow, idxbrow, impa, impb, w2row, hvrow,
               featb, hrows, wbuf):
        c = lax.axis_index("c")
        s = lax.axis_index("s")
        wid = c * _NS + s
        i16 = _iota16()

        for q in range(_FILLS // _L):
            fillv[pl.ds(q * _L, _L)] = jnp.full((_L,), -1, jnp.int32)
        base = s * span

        def memset_body(q, _):
            pltpu.sync_copy(fillv, table.at[pl.ds(base + q * _FILLS, _FILLS)])
            return _
        lax.fori_loop(0, span // _FILLS, memset_body, None)

        pltpu.sync_copy(widx_h.at[s], widx_v)
        jbase = s * wpt
        for k in range(wk):
            for v in range(_CHUNK // _L):
                jval_v[k, pl.ds(v * _L, _L)] = i16 + (jbase + k * _CHUNK + v * _L)

        plsc.subcore_barrier()

        for k in range(wk):
            pltpu.sync_copy(jval_v.at[k], table.at[widx_v.at[k]])

        for _r in range(_FIX_ROUNDS):
            plsc.subcore_barrier()

            def fix_body(k, _):
                pltpu.sync_copy(table.at[widx_v.at[k]], tvrow)
                for v in range(_CHUNK // _L):
                    cidx = i16 + v * _L
                    jv = plsc.load_gather(jval_v, [_splat(k), cidx])
                    wv = plsc.load_gather(widx_v, [_splat(k), cidx])
                    tvv = tvrow[pl.ds(v * _L, _L)]
                    fixrow[pl.ds(v * _L, _L)] = jnp.where(tvv < jv, wv, dummy)
                pltpu.sync_copy(jval_v.at[k], table.at[fixrow])
                return _
            lax.fori_loop(0, wk, fix_body, None)

        plsc.subcore_barrier()

        def chunk_body(k, _):
            pltpu.sync_copy(sidx_h.at[wid, k], srow)
            pltpu.sync_copy(table.at[srow], jwrow)
            pltpu.sync_copy(mimp.at[srow], impa)
            for v in range(_CHUNK // _L):
                sl = pl.ds(v * _L, _L)
                jw = jwrow[sl]
                idxbrow[sl] = jnp.where(jw >= 0, jw, 0)
            pltpu.sync_copy(feats.at[idxbrow], featb)
            pltpu.sync_copy(fimp.at[idxbrow], impb)
            for v in range(_CHUNK // _L):
                sl = pl.ds(v * _L, _L)
                hit = jwrow[sl] >= 0
                w2row[sl] = jnp.where(hit, 0.0, impa[sl])
                hvrow[sl] = jnp.where(hit, impb[sl], 0.0)

            def row_body(i, _):
                si = _splat(i)
                w16 = plsc.load_gather(w2row, [si])
                h16 = plsc.load_gather(hvrow, [si])
                for ccol in range(D // _L):
                    cidx = i16 + ccol * _L
                    b = plsc.load_gather(featb, [si, cidx])
                    plsc.store_scatter(hrows, [si, cidx], b * h16)
                    plsc.store_scatter(wbuf, [si, cidx], w16)
                return _
            lax.fori_loop(0, _CHUNK, row_body, None)

            rbase = wid * spw + k * _CHUNK
            pltpu.sync_copy(hrows, hit_h.at[pl.ds(rbase, _CHUNK)])
            pltpu.sync_copy(wbuf, wexp_h.at[pl.ds(rbase, _CHUNK)])
            return _
        lax.fori_loop(0, sk, chunk_body, None)

    return stage1


def _make_stage2(M, D, S):
    spw = S // (_NC * _NS)
    sk = spw // _CHUNK
    mesh = plsc.VectorSubcoreMesh(core_axis_name="c", subcore_axis_name="s")

    @functools.partial(
        pl.kernel,
        mesh=mesh,
        out_type=jax.ShapeDtypeStruct((S, D), jnp.float32),
        scratch_types=[
            pltpu.VMEM((_CHUNK,), jnp.int32),
            pltpu.VMEM((_CHUNK, D), jnp.float32),
            pltpu.VMEM((_CHUNK, D), jnp.float32),
            pltpu.VMEM((_CHUNK, D), jnp.float32),
            pltpu.SemaphoreType.DMA,
        ],
        compiler_params=_PARAMS,
    )
    def stage2(mem, sidx_h, hit_h, wexp_h, out, srow, mema, hr, wx, sem):
        c = lax.axis_index("c")
        s = lax.axis_index("s")
        wid = c * _NS + s

        def chunk_body(k, _):
            rbase = wid * spw + k * _CHUNK
            pltpu.sync_copy(sidx_h.at[wid, k], srow)
            cpa = pltpu.async_copy(mem.at[srow], mema, sem)
            cph = pltpu.async_copy(hit_h.at[pl.ds(rbase, _CHUNK)], hr, sem)
            cpw = pltpu.async_copy(wexp_h.at[pl.ds(rbase, _CHUNK)], wx, sem)
            cpa.wait()
            cph.wait()
            cpw.wait()
            for r in range(_CHUNK):
                for ccol in range(0, D, _L):
                    sl = pl.ds(ccol, _L)
                    mema[r, sl] = mema[r, sl] * wx[r, sl] + hr[r, sl]
            pltpu.sync_copy(mema, out.at[pl.ds(rbase, _CHUNK)])
            return _
        lax.fori_loop(0, sk, chunk_body, None)

    return stage2


def kernel(memory_features, memory_importance, features, importance,
           write_idx, sample_idx):
    M, D = memory_features.shape
    B = write_idx.shape[0]
    S = sample_idx.shape[0]
    widx3 = write_idx.reshape(_NS, B // (_NS * _CHUNK), _CHUNK)
    sidx3 = sample_idx.reshape(_NC * _NS, S // (_NC * _NS * _CHUNK), _CHUNK)
    hitrows, wexp = _make_stage1(M, D, B, S)(
        memory_importance, features, importance, widx3, sidx3)
    return _make_stage2(M, D, S)(memory_features, sidx3, hitrows, wexp)
